# baseline (speedup 1.0000x reference)
import jax
import jax.numpy as jnp
from jax import lax
from jax.experimental import pallas as pl
from jax.experimental.pallas import tpu as pltpu
from jax.experimental.pallas import tpu_sc as plsc

OBS = 64
SEQ_LEN = 200
BATCH = 4096
OUT_COLS = SEQ_LEN * OBS + 128
L = 16
OCH = 100
CCH = OCH * OBS
NCHUNK = 32
NBUF = 2


def _sc_kernel(seq, skill, obj_idx, obj_embed, out,
               idxv, embv, skbuf, buf0, buf1,
               semi0, semi1, semo0, semo1, sems):
    w = lax.axis_index("s") * 2 + lax.axis_index("c")
    bufs = [buf0, buf1]
    sem_in = [semi0, semi1]
    sem_out = [semo0, semo1]

    pltpu.sync_copy(obj_idx, idxv.at[pl.ds(0, BATCH)])
    pltpu.sync_copy(obj_embed, embv)
    evecs = [embv[pl.ds(L * j, L)] for j in range(OBS // L)]

    b0 = pl.multiple_of(w * 128, 128)
    skill_in = pltpu.make_async_copy(
        skill.at[pl.ds(b0, 128), :], skbuf, sems)
    skill_in.start()

    def rg(t):
        r_ = pl.multiple_of((w * 16 + t // 2) * 8, 8)
        g_ = t % 2
        return r_, g_

    def start_in(kb, t):
        r_, g_ = rg(t)
        pltpu.make_async_copy(
            seq.at[pl.ds(r_, 8),
                   pl.ds(pl.multiple_of(CCH * g_, 128), CCH)],
            bufs[kb], sem_in[kb]).start()

    def wait_in(kb):
        pltpu.make_async_copy(
            seq.at[pl.ds(0, 8), pl.ds(0, CCH)], bufs[kb],
            sem_in[kb]).wait()

    def start_out(kb, t):
        r_, g_ = rg(t)
        pltpu.make_async_copy(
            bufs[kb],
            out.at[pl.ds(r_, 8),
                   pl.ds(pl.multiple_of(CCH * g_, 128), CCH)],
            sem_out[kb]).start()

    def wait_out(kb):
        pltpu.make_async_copy(
            bufs[kb],
            out.at[pl.ds(0, 8), pl.ds(0, CCH)], sem_out[kb]).wait()

    def apply_embed(kb, t):
        r_, g_ = rg(t)
        iv = idxv[pl.ds(r_, L)]
        o_lo = OCH * g_
        for r in range(8):
            o_b = iv[r]
            c0 = OBS * (o_b - o_lo)
            hit = jnp.logical_and(o_b >= o_lo, o_b < o_lo + OCH)

            @pl.when(hit)
            def _():
                for j in range(OBS // L):
                    bufs[kb][r, pl.ds(c0 + L * j, L)] = (
                        bufs[kb][r, pl.ds(c0 + L * j, L)] + evecs[j])

    def step(kb, t):
        wait_in(kb)
        apply_embed(kb, t)
        start_out(kb, t)
        wait_out(kb)

        @pl.when(t + NBUF < NCHUNK)
        def _():
            start_in(kb, t + NBUF)

    for kb in range(NBUF):
        start_in(kb, kb)

    def body(i, carry):
        for kb in range(NBUF):
            step(kb, NBUF * i + kb)
        return carry

    niter = NCHUNK // NBUF
    lax.fori_loop(0, niter, body, 0)

    skill_in.wait()
    pltpu.sync_copy(
        skbuf, out.at[pl.ds(b0, 128), pl.ds(SEQ_LEN * OBS, 128)])


@jax.jit
def kernel(seq, skill, obj_idx, obj_embed):
    obj_idx = obj_idx.astype(jnp.int32)
    seq = seq.reshape(BATCH, SEQ_LEN * OBS)
    mesh = plsc.VectorSubcoreMesh(core_axis_name="c", subcore_axis_name="s")
    dma = pltpu.SemaphoreType.DMA
    out = pl.kernel(
        _sc_kernel,
        out_type=jax.ShapeDtypeStruct((BATCH, OUT_COLS), jnp.float32),
        mesh=mesh,
        scratch_types=[
            pltpu.VMEM((BATCH + L,), jnp.int32),
            pltpu.VMEM((OBS,), jnp.float32),
            pltpu.VMEM((128, 128), jnp.float32),
            pltpu.VMEM((8, CCH), jnp.float32),
            pltpu.VMEM((8, CCH), jnp.float32),
            dma, dma, dma, dma, dma,
        ],
    )(seq, skill, obj_idx, obj_embed)
    return out


# speedup vs baseline: 1.5758x; 1.5758x over previous
import jax
import jax.numpy as jnp
from jax import lax
from jax.experimental import pallas as pl
from jax.experimental.pallas import tpu as pltpu
from jax.experimental.pallas import tpu_sc as plsc

OBS = 64
SEQ_LEN = 200
BATCH = 4096
SEQ_COLS = SEQ_LEN * OBS
OUT_COLS = SEQ_COLS + 128
L = 16
TCB = 256
TCC = 1280
ROW_T = OUT_COLS // 128
N_SLAB = BATCH // 8
RAW_ROWS = N_SLAB * ROW_T * 8


def _tc_transpose(seq_ref, out_ref):
    out_ref[...] = seq_ref[...].T


def _sc_update(skill, obj_idx, obj_embed, o4,
               idxv, hv, qv, q2v, embv, rows, skb, semg):
    w = lax.axis_index("s") * 2 + lax.axis_index("c")
    b0 = pl.multiple_of(w * 128, 128)
    lanes = lax.iota(jnp.int32, L)

    pltpu.sync_copy(obj_idx, idxv.at[pl.ds(0, BATCH)])
    pltpu.sync_copy(obj_embed, embv)
    pltpu.sync_copy(skill.at[pl.ds(b0, 128), :], skb)
    evecs = [embv[pl.ds(L * j, L)] for j in range(OBS // L)]

    for m in range(8):
        b = b0 + L * m + lanes
        iv = idxv[pl.ds(b0 + L * m, L)]
        slab = lax.shift_right_logical(b, 3) * ROW_T
        r = lax.bitwise_and(b, 7)
        qv[pl.ds(L * m, L)] = (slab + lax.shift_right_logical(iv, 1)) * 8 + r
        q2v[pl.ds(L * m, L)] = (slab + SEQ_COLS // 128) * 8 + r
        hv[pl.ds(L * m, L)] = lax.bitwise_and(iv, 1)

    pltpu.async_copy(o4.at[qv], rows, semg).wait()

    def mod16(m, carry):
        hvec = hv[pl.ds(L * m, L)]
        for r in range(L):
            h_j = hvec[r]
            j = L * m + r
            for v in range(8):
                gate = jnp.where(h_j == v // 4, 1.0, 0.0)
                rows[j, pl.ds(L * v, L)] = (
                    rows[j, pl.ds(L * v, L)] + evecs[v % 4] * gate)
        return carry

    lax.fori_loop(0, 8, mod16, 0)

    pltpu.async_copy(rows, o4.at[qv], semg).wait()
    pltpu.async_copy(skb, o4.at[q2v], semg).wait()


@jax.jit
def kernel(seq, skill, obj_idx, obj_embed):
    obj_idx = obj_idx.astype(jnp.int32)
    seq_v = seq.reshape(BATCH, SEQ_COLS).T

    out = pl.pallas_call(
        _tc_transpose,
        grid=(BATCH // TCB, SEQ_COLS // TCC),
        in_specs=[pl.BlockSpec((TCC, TCB), lambda i, j: (j, i))],
        out_specs=pl.BlockSpec((TCB, TCC), lambda i, j: (i, j)),
        out_shape=jax.ShapeDtypeStruct((BATCH, OUT_COLS), jnp.float32),
    )(seq_v)

    o4 = (out.reshape(N_SLAB, 8, ROW_T, 128).transpose(0, 2, 1, 3)
          .reshape(RAW_ROWS, 128))
    ref = jax.new_ref(o4)
    mesh = plsc.VectorSubcoreMesh(core_axis_name="c", subcore_axis_name="s")
    pl.kernel(
        _sc_update,
        out_type=(),
        mesh=mesh,
        scratch_types=[
            pltpu.VMEM((BATCH,), jnp.int32),
            pltpu.VMEM((128,), jnp.int32),
            pltpu.VMEM((128,), jnp.int32),
            pltpu.VMEM((128,), jnp.int32),
            pltpu.VMEM((OBS,), jnp.float32),
            pltpu.VMEM((128, 128), jnp.float32),
            pltpu.VMEM((128, 128), jnp.float32),
            pltpu.SemaphoreType.DMA,
        ],
    )(skill, obj_idx, obj_embed, ref)
    res = ref[...]
    return (res.reshape(N_SLAB, ROW_T, 8, 128).transpose(0, 2, 1, 3)
            .reshape(BATCH, OUT_COLS))


# speedup vs baseline: 2.0338x; 1.2906x over previous
import jax
import jax.numpy as jnp
from jax import lax
from jax.experimental import pallas as pl
from jax.experimental.pallas import tpu as pltpu
from jax.experimental.pallas import tpu_sc as plsc

OBS = 64
SEQ_LEN = 200
BATCH = 4096
SEQ_COLS = SEQ_LEN * OBS
OUT_COLS = SEQ_COLS + 128
L = 16
TCB = 512
TCC = 1280
ROW_T = OUT_COLS // 128
N_SLAB = BATCH // 8
RAW_ROWS = N_SLAB * ROW_T * 8


def _tc_transpose(seq_ref, out_ref):
    out_ref[...] = seq_ref[...].T


def _sc_update(skill, obj_idx, obj_embed, o4,
               idxv, hv, qv, q2v, embv, rows, skb, semg):
    w = lax.axis_index("s") * 2 + lax.axis_index("c")
    b0 = pl.multiple_of(w * 128, 128)
    lanes = lax.iota(jnp.int32, L)

    pltpu.sync_copy(obj_idx, idxv.at[pl.ds(0, BATCH)])
    pltpu.sync_copy(obj_embed, embv)
    pltpu.sync_copy(skill.at[pl.ds(b0, 128), :], skb)
    evecs = [embv[pl.ds(L * j, L)] for j in range(OBS // L)]

    for m in range(8):
        b = b0 + L * m + lanes
        iv = idxv[pl.ds(b0 + L * m, L)]
        slab = lax.shift_right_logical(b, 3) * ROW_T
        r = lax.bitwise_and(b, 7)
        qv[pl.ds(L * m, L)] = (slab + lax.shift_right_logical(iv, 1)) * 8 + r
        q2v[pl.ds(L * m, L)] = (slab + SEQ_COLS // 128) * 8 + r
        hv[pl.ds(L * m, L)] = lax.bitwise_and(iv, 1)

    pltpu.async_copy(o4.at[qv], rows, semg).wait()

    def mod16(m, carry):
        hvec = hv[pl.ds(L * m, L)]
        for r in range(L):
            h_j = hvec[r]
            j = L * m + r
            for v in range(8):
                gate = jnp.where(h_j == v // 4, 1.0, 0.0)
                rows[j, pl.ds(L * v, L)] = (
                    rows[j, pl.ds(L * v, L)] + evecs[v % 4] * gate)
        return carry

    lax.fori_loop(0, 8, mod16, 0)

    pltpu.async_copy(rows, o4.at[qv], semg).wait()
    pltpu.async_copy(skb, o4.at[q2v], semg).wait()


@jax.jit
def kernel(seq, skill, obj_idx, obj_embed):
    obj_idx = obj_idx.astype(jnp.int32)
    seq_v = seq.reshape(BATCH, SEQ_COLS).T

    out = pl.pallas_call(
        _tc_transpose,
        grid=(BATCH // TCB, SEQ_COLS // TCC),
        in_specs=[pl.BlockSpec((TCC, TCB), lambda i, j: (j, i))],
        out_specs=pl.BlockSpec((TCB, TCC), lambda i, j: (i, j)),
        out_shape=jax.ShapeDtypeStruct((BATCH, OUT_COLS), jnp.float32),
    )(seq_v)

    o4 = (out.reshape(N_SLAB, 8, ROW_T, 128).transpose(0, 2, 1, 3)
          .reshape(RAW_ROWS, 128))
    ref = jax.new_ref(o4)
    mesh = plsc.VectorSubcoreMesh(core_axis_name="c", subcore_axis_name="s")
    pl.kernel(
        _sc_update,
        out_type=(),
        mesh=mesh,
        scratch_types=[
            pltpu.VMEM((BATCH,), jnp.int32),
            pltpu.VMEM((128,), jnp.int32),
            pltpu.VMEM((128,), jnp.int32),
            pltpu.VMEM((128,), jnp.int32),
            pltpu.VMEM((OBS,), jnp.float32),
            pltpu.VMEM((128, 128), jnp.float32),
            pltpu.VMEM((128, 128), jnp.float32),
            pltpu.SemaphoreType.DMA,
        ],
    )(skill, obj_idx, obj_embed, ref)
    res = ref[...]
    return (res.reshape(N_SLAB, ROW_T, 8, 128).transpose(0, 2, 1, 3)
            .reshape(BATCH, OUT_COLS))


# speedup vs baseline: 2.2163x; 1.0898x over previous
import jax
import jax.numpy as jnp
from jax import lax
from jax.experimental import pallas as pl
from jax.experimental.pallas import tpu as pltpu
from jax.experimental.pallas import tpu_sc as plsc

OBS = 64
SEQ_LEN = 200
BATCH = 4096
SEQ_COLS = SEQ_LEN * OBS
OUT_COLS = SEQ_COLS + 128
L = 16
TCB = 1024
TCC = 1280
ROW_T = OUT_COLS // 128
N_SLAB = BATCH // 8
RAW_ROWS = N_SLAB * ROW_T * 8


def _tc_transpose(seq_ref, out_ref):
    out_ref[...] = seq_ref[...].T


def _sc_update(skill, obj_idx, obj_embed, o4,
               idxv, hv, qv, q2v, embv, rows, skb, semg):
    w = lax.axis_index("s") * 2 + lax.axis_index("c")
    b0 = pl.multiple_of(w * 128, 128)
    lanes = lax.iota(jnp.int32, L)

    pltpu.sync_copy(obj_idx, idxv.at[pl.ds(0, BATCH)])
    pltpu.sync_copy(obj_embed, embv)
    pltpu.sync_copy(skill.at[pl.ds(b0, 128), :], skb)
    evecs = [embv[pl.ds(L * j, L)] for j in range(OBS // L)]

    for m in range(8):
        b = b0 + L * m + lanes
        iv = idxv[pl.ds(b0 + L * m, L)]
        slab = lax.shift_right_logical(b, 3) * ROW_T
        r = lax.bitwise_and(b, 7)
        qv[pl.ds(L * m, L)] = (slab + lax.shift_right_logical(iv, 1)) * 8 + r
        q2v[pl.ds(L * m, L)] = (slab + SEQ_COLS // 128) * 8 + r
        hv[pl.ds(L * m, L)] = lax.bitwise_and(iv, 1)

    pltpu.async_copy(o4.at[qv], rows, semg).wait()

    def mod16(m, carry):
        hvec = hv[pl.ds(L * m, L)]
        for r in range(L):
            h_j = hvec[r]
            j = L * m + r
            for v in range(8):
                gate = jnp.where(h_j == v // 4, 1.0, 0.0)
                rows[j, pl.ds(L * v, L)] = (
                    rows[j, pl.ds(L * v, L)] + evecs[v % 4] * gate)
        return carry

    lax.fori_loop(0, 8, mod16, 0)

    pltpu.async_copy(rows, o4.at[qv], semg).wait()
    pltpu.async_copy(skb, o4.at[q2v], semg).wait()


@jax.jit
def kernel(seq, skill, obj_idx, obj_embed):
    obj_idx = obj_idx.astype(jnp.int32)
    seq_v = seq.reshape(BATCH, SEQ_COLS).T

    out = pl.pallas_call(
        _tc_transpose,
        grid=(BATCH // TCB, SEQ_COLS // TCC),
        in_specs=[pl.BlockSpec((TCC, TCB), lambda i, j: (j, i))],
        out_specs=pl.BlockSpec((TCB, TCC), lambda i, j: (i, j)),
        out_shape=jax.ShapeDtypeStruct((BATCH, OUT_COLS), jnp.float32),
    )(seq_v)

    o4 = (out.reshape(N_SLAB, 8, ROW_T, 128).transpose(0, 2, 1, 3)
          .reshape(RAW_ROWS, 128))
    ref = jax.new_ref(o4)
    mesh = plsc.VectorSubcoreMesh(core_axis_name="c", subcore_axis_name="s")
    pl.kernel(
        _sc_update,
        out_type=(),
        mesh=mesh,
        scratch_types=[
            pltpu.VMEM((BATCH,), jnp.int32),
            pltpu.VMEM((128,), jnp.int32),
            pltpu.VMEM((128,), jnp.int32),
            pltpu.VMEM((128,), jnp.int32),
            pltpu.VMEM((OBS,), jnp.float32),
            pltpu.VMEM((128, 128), jnp.float32),
            pltpu.VMEM((128, 128), jnp.float32),
            pltpu.SemaphoreType.DMA,
        ],
    )(skill, obj_idx, obj_embed, ref)
    res = ref[...]
    return (res.reshape(N_SLAB, ROW_T, 8, 128).transpose(0, 2, 1, 3)
            .reshape(BATCH, OUT_COLS))


# speedup vs baseline: 2.2433x; 1.0122x over previous
import jax
import jax.numpy as jnp
from jax import lax
from jax.experimental import pallas as pl
from jax.experimental.pallas import tpu as pltpu
from jax.experimental.pallas import tpu_sc as plsc

OBS = 64
SEQ_LEN = 200
BATCH = 4096
SEQ_COLS = SEQ_LEN * OBS
OUT_COLS = SEQ_COLS + 128
L = 16
TCB = 2048
TCC = 1280
ROW_T = OUT_COLS // 128
N_SLAB = BATCH // 8
RAW_ROWS = N_SLAB * ROW_T * 8


def _tc_transpose(seq_ref, out_ref):
    out_ref[...] = seq_ref[...].T


def _sc_update(skill, obj_idx, obj_embed, o4,
               idxv, hv, qv, q2v, embv, rows, skb, semg):
    w = lax.axis_index("s") * 2 + lax.axis_index("c")
    b0 = pl.multiple_of(w * 128, 128)
    lanes = lax.iota(jnp.int32, L)

    pltpu.sync_copy(obj_idx, idxv.at[pl.ds(0, BATCH)])
    pltpu.sync_copy(obj_embed, embv)
    pltpu.sync_copy(skill.at[pl.ds(b0, 128), :], skb)
    evecs = [embv[pl.ds(L * j, L)] for j in range(OBS // L)]

    for m in range(8):
        b = b0 + L * m + lanes
        iv = idxv[pl.ds(b0 + L * m, L)]
        slab = lax.shift_right_logical(b, 3) * ROW_T
        r = lax.bitwise_and(b, 7)
        qv[pl.ds(L * m, L)] = (slab + lax.shift_right_logical(iv, 1)) * 8 + r
        q2v[pl.ds(L * m, L)] = (slab + SEQ_COLS // 128) * 8 + r
        hv[pl.ds(L * m, L)] = lax.bitwise_and(iv, 1)

    pltpu.async_copy(o4.at[qv], rows, semg).wait()

    def mod16(m, carry):
        hvec = hv[pl.ds(L * m, L)]
        for r in range(L):
            h_j = hvec[r]
            j = L * m + r
            for v in range(8):
                gate = jnp.where(h_j == v // 4, 1.0, 0.0)
                rows[j, pl.ds(L * v, L)] = (
                    rows[j, pl.ds(L * v, L)] + evecs[v % 4] * gate)
        return carry

    lax.fori_loop(0, 8, mod16, 0)

    pltpu.async_copy(rows, o4.at[qv], semg).wait()
    pltpu.async_copy(skb, o4.at[q2v], semg).wait()


@jax.jit
def kernel(seq, skill, obj_idx, obj_embed):
    obj_idx = obj_idx.astype(jnp.int32)
    seq_v = seq.reshape(BATCH, SEQ_COLS).T

    out = pl.pallas_call(
        _tc_transpose,
        grid=(BATCH // TCB, SEQ_COLS // TCC),
        in_specs=[pl.BlockSpec((TCC, TCB), lambda i, j: (j, i))],
        out_specs=pl.BlockSpec((TCB, TCC), lambda i, j: (i, j)),
        out_shape=jax.ShapeDtypeStruct((BATCH, OUT_COLS), jnp.float32),
    )(seq_v)

    o4 = (out.reshape(N_SLAB, 8, ROW_T, 128).transpose(0, 2, 1, 3)
          .reshape(RAW_ROWS, 128))
    ref = jax.new_ref(o4)
    mesh = plsc.VectorSubcoreMesh(core_axis_name="c", subcore_axis_name="s")
    pl.kernel(
        _sc_update,
        out_type=(),
        mesh=mesh,
        scratch_types=[
            pltpu.VMEM((BATCH,), jnp.int32),
            pltpu.VMEM((128,), jnp.int32),
            pltpu.VMEM((128,), jnp.int32),
            pltpu.VMEM((128,), jnp.int32),
            pltpu.VMEM((OBS,), jnp.float32),
            pltpu.VMEM((128, 128), jnp.float32),
            pltpu.VMEM((128, 128), jnp.float32),
            pltpu.SemaphoreType.DMA,
        ],
    )(skill, obj_idx, obj_embed, ref)
    res = ref[...]
    return (res.reshape(N_SLAB, ROW_T, 8, 128).transpose(0, 2, 1, 3)
            .reshape(BATCH, OUT_COLS))
